# baseline (speedup 1.0000x reference)
import functools

import jax
import jax.numpy as jnp
from jax import lax
from jax.experimental import pallas as pl
from jax.experimental.pallas import tpu as pltpu
from jax.experimental.pallas import tpu_sc as plsc

BATCH = 16384
EMB_DIM = 64
NUM_WORKERS = 32
ROWS_PER_WORKER = BATCH // NUM_WORKERS
CHUNK = 128
NUM_CHUNKS = ROWS_PER_WORKER // CHUNK
LANES = 16
NUM_GROUPS = ROWS_PER_WORKER // LANES


def _sc_body(uid_h, sid_h, pid_h, u_emb_h, v_emb_h, w_h, b_h,
             score_h, pol_h,
             uid_v, sid_v, pid_v, u_rows, v_rows, p_rows,
             score_v, pol_v, w_v, b_v,
             sem_idx, sem_u, sem_v, sem_p, sem_w):
    wid = lax.axis_index("s") * 2 + lax.axis_index("c")
    base = wid * ROWS_PER_WORKER

    idx_handles = []
    for j in range(NUM_CHUNKS):
        off = base + j * CHUNK
        idx_handles.append(pltpu.async_copy(
            uid_h.at[pl.ds(off, CHUNK)], uid_v.at[j], sem_idx))
        idx_handles.append(pltpu.async_copy(
            sid_h.at[pl.ds(off, CHUNK)], sid_v.at[j], sem_idx))
        idx_handles.append(pltpu.async_copy(
            pid_h.at[pl.ds(off, CHUNK)], pid_v.at[j], sem_idx))
    w_handle = pltpu.async_copy(w_h, w_v, sem_w)
    b_handle = pltpu.async_copy(b_h, b_v, sem_w)
    for h in idx_handles:
        h.wait()

    row_handles = []
    for j in range(NUM_CHUNKS):
        dst = pl.ds(j * CHUNK, CHUNK)
        row_handles.append(pltpu.async_copy(
            u_emb_h.at[uid_v.at[j]], u_rows.at[dst], sem_u))
        row_handles.append(pltpu.async_copy(
            v_emb_h.at[sid_v.at[j]], v_rows.at[dst], sem_v))
        row_handles.append(pltpu.async_copy(
            u_emb_h.at[pid_v.at[j]], p_rows.at[dst], sem_p))
    w_handle.wait()
    b_handle.wait()
    for h in row_handles:
        h.wait()

    bias = b_v[...]
    zeros = jnp.zeros((LANES,), jnp.float32)
    lane_iota = lax.iota(jnp.int32, LANES)

    def group_body(g, carry):
        rows = g * LANES + lane_iota

        def col_body(c, accs):
            acc_s, acc_p = accs
            cvec = jnp.full((LANES,), 0, jnp.int32) + c
            uu = plsc.load_gather(u_rows, [rows, cvec])
            vv = plsc.load_gather(v_rows, [rows, cvec])
            pp = plsc.load_gather(p_rows, [rows, cvec])
            wc = plsc.load_gather(w_v, [cvec])
            return acc_s + uu * vv, acc_p + pp * wc

        acc_s, acc_p = lax.fori_loop(0, EMB_DIM, col_body, (zeros, zeros))
        out_slice = pl.ds(pl.multiple_of(g * LANES, LANES), LANES)
        score_v[out_slice] = 1.0 / (1.0 + jnp.exp(-acc_s))
        pol_v[out_slice] = 1.0 / (1.0 + jnp.exp(-(acc_p + bias)))
        return carry

    lax.fori_loop(0, NUM_GROUPS, group_body, 0)

    pltpu.sync_copy(score_v, score_h.at[pl.ds(base, ROWS_PER_WORKER)])
    pltpu.sync_copy(pol_v, pol_h.at[pl.ds(base, ROWS_PER_WORKER)])


@jax.jit
def _run(user_id, subreddit_id, political_user_ids, u_emb, v_emb, w, b16):
    mesh = plsc.VectorSubcoreMesh(core_axis_name="c", subcore_axis_name="s")
    f32 = jnp.float32
    call = functools.partial(
        pl.kernel,
        mesh=mesh,
        out_type=[
            jax.ShapeDtypeStruct((BATCH,), f32),
            jax.ShapeDtypeStruct((BATCH,), f32),
        ],
        scratch_types=[
            pltpu.VMEM((NUM_CHUNKS, CHUNK), jnp.int32),
            pltpu.VMEM((NUM_CHUNKS, CHUNK), jnp.int32),
            pltpu.VMEM((NUM_CHUNKS, CHUNK), jnp.int32),
            pltpu.VMEM((ROWS_PER_WORKER, EMB_DIM), f32),
            pltpu.VMEM((ROWS_PER_WORKER, EMB_DIM), f32),
            pltpu.VMEM((ROWS_PER_WORKER, EMB_DIM), f32),
            pltpu.VMEM((ROWS_PER_WORKER,), f32),
            pltpu.VMEM((ROWS_PER_WORKER,), f32),
            pltpu.VMEM((EMB_DIM,), f32),
            pltpu.VMEM((LANES,), f32),
            pltpu.SemaphoreType.DMA,
            pltpu.SemaphoreType.DMA,
            pltpu.SemaphoreType.DMA,
            pltpu.SemaphoreType.DMA,
            pltpu.SemaphoreType.DMA,
        ],
        compiler_params=pltpu.CompilerParams(
            needs_layout_passes=False, use_tc_tiling_on_sc=False),
    )
    return call(_sc_body)(user_id, subreddit_id, political_user_ids,
                          u_emb, v_emb, w, b16)


def kernel(user_id, subreddit_id, political_user_ids, u_emb, v_emb, pol_W, pol_b):
    w = pol_W.reshape(EMB_DIM)
    b16 = jnp.broadcast_to(pol_b, (LANES,))
    score, pol = _run(user_id.astype(jnp.int32), subreddit_id.astype(jnp.int32),
                      political_user_ids.astype(jnp.int32), u_emb, v_emb, w, b16)
    return score, pol.reshape(BATCH, 1)
